# Initial kernel scaffold; baseline (speedup 1.0000x reference)
import jax
import jax.numpy as jnp
from jax.experimental import pallas as pl


def kernel(x, edge_index, edge_attr, params):
    raise NotImplementedError("write your pallas kernel here")


# speedup vs baseline: 2.8312x; 2.8312x over previous
import functools

import jax
import jax.numpy as jnp
from jax import lax
from jax.experimental import pallas as pl
from jax.experimental.pallas import tpu as pltpu
from jax.experimental.pallas import tpu_sc as plsc

NC = 2
NS = 16
CHUNK = 128


def _make_edge_aggr(n_nodes, n_edges, d):
    nchunk = n_edges // CHUNK
    nfull, extra = nchunk // (NC * NS), nchunk % (NC * NS)
    n_pad = ((n_nodes + NS * CHUNK - 1) // (NS * CHUNK)) * (NS * CHUNK)
    rows_per_tile = n_pad // NS
    rstep = CHUNK
    q = rows_per_tile // rstep

    mesh = plsc.VectorSubcoreMesh(
        core_axis_name="c", subcore_axis_name="s", num_cores=NC, num_subcores=NS)

    @functools.partial(
        pl.kernel,
        mesh=mesh,
        compiler_params=pltpu.CompilerParams(use_tc_tiling_on_sc=False),
        out_type=jax.ShapeDtypeStruct((NC * n_pad, d), jnp.float32),
        scratch_types=[
            pltpu.VMEM((CHUNK,), jnp.int32),
            pltpu.VMEM((CHUNK,), jnp.int32),
            pltpu.VMEM((CHUNK, d), jnp.float32),
            pltpu.VMEM((CHUNK, d), jnp.float32),
            pltpu.VMEM_SHARED((n_pad, d), jnp.float32),
            pltpu.SemaphoreType.DMA,
        ],
    )
    def edge_kernel(h_hbm, src_hbm, dst_hbm, e_hbm, out_hbm,
                    src_v, dst_v, rows_v, e_v, aggr_sh, sem):
        c = lax.axis_index("c")
        s = lax.axis_index("s")
        w = c * NS + s

        def zbody(i, _):
            for v in range(d // 16):
                e_v[i, pl.ds(v * 16, 16)] = jnp.zeros((16,), jnp.float32)
            return 0
        lax.fori_loop(0, CHUNK, zbody, 0)
        base = s * rows_per_tile
        for qq in range(q):
            pltpu.sync_copy(e_v.at[pl.ds(0, rstep)],
                            aggr_sh.at[pl.ds(base + qq * rstep, rstep)])
        plsc.subcore_barrier()

        n_my = jnp.where(w < extra, nfull + 1, nfull)

        def body(i, _):
            j = w + (NC * NS) * i
            pltpu.sync_copy(src_hbm.at[j], src_v)
            pltpu.sync_copy(dst_hbm.at[j], dst_v)
            pltpu.sync_copy(e_hbm.at[pl.ds(j * CHUNK, CHUNK)], e_v)
            pltpu.async_copy(h_hbm.at[src_v], rows_v, sem).wait()

            def vbody(ii, _):
                for v in range(d // 16):
                    sl = pl.ds(v * 16, 16)
                    rows_v[ii, sl] = jnp.maximum(rows_v[ii, sl] + e_v[ii, sl], 0.0)
                return 0
            lax.fori_loop(0, CHUNK, vbody, 0)

            pltpu.sync_copy(rows_v, aggr_sh.at[dst_v], add=True)
            return 0
        lax.fori_loop(0, n_my, body, 0)

        plsc.subcore_barrier()
        for qq in range(q):
            r0 = base + qq * rstep
            pltpu.sync_copy(aggr_sh.at[pl.ds(r0, rstep)],
                            out_hbm.at[pl.ds(c * n_pad + r0, rstep)])

    return edge_kernel


def _edge_bias(edge_attr, we, be):
    e_total, k = edge_attr.shape
    d = we.shape[1]
    be2 = be.reshape(1, d)
    blk = 4000

    def body(a_ref, w_ref, b_ref, o_ref):
        o_ref[...] = jnp.dot(a_ref[...], w_ref[...],
                             preferred_element_type=jnp.float32, precision=lax.Precision.HIGHEST) + b_ref[...]

    return pl.pallas_call(
        body,
        grid=(e_total // blk,),
        in_specs=[
            pl.BlockSpec((blk, k), lambda i: (i, 0)),
            pl.BlockSpec((k, d), lambda i: (0, 0)),
            pl.BlockSpec((1, d), lambda i: (0, 0)),
        ],
        out_specs=pl.BlockSpec((blk, d), lambda i: (i, 0)),
        out_shape=jax.ShapeDtypeStruct((e_total, d), jnp.float32),
    )(edge_attr, we, be2)


def _node_mlp(h, a0, a1, w1, b1, w2, b2):
    n, din = h.shape
    dh = w1.shape[1]
    blk = 1000

    def body(h_ref, a0_ref, a1_ref, w1_ref, b1_ref, w2_ref, b2_ref, o_ref):
        z = h_ref[...] + a0_ref[...] + a1_ref[...]
        t = jnp.maximum(jnp.dot(z, w1_ref[...],
                                preferred_element_type=jnp.float32, precision=lax.Precision.HIGHEST) + b1_ref[...], 0.0)
        o_ref[...] = jnp.maximum(jnp.dot(t, w2_ref[...],
                                         preferred_element_type=jnp.float32, precision=lax.Precision.HIGHEST) + b2_ref[...], 0.0)

    return pl.pallas_call(
        body,
        grid=(n // blk,),
        in_specs=[
            pl.BlockSpec((blk, din), lambda i: (i, 0)),
            pl.BlockSpec((blk, din), lambda i: (i, 0)),
            pl.BlockSpec((blk, din), lambda i: (i, 0)),
            pl.BlockSpec((din, dh), lambda i: (0, 0)),
            pl.BlockSpec((1, dh), lambda i: (0, 0)),
            pl.BlockSpec((dh, dh), lambda i: (0, 0)),
            pl.BlockSpec((1, dh), lambda i: (0, 0)),
        ],
        out_specs=pl.BlockSpec((blk, dh), lambda i: (i, 0)),
        out_shape=jax.ShapeDtypeStruct((n, dh), jnp.float32),
    )(h, a0, a1, w1, b1.reshape(1, dh), w2, b2.reshape(1, dh))


def _head(h, w1, b1, w2, b2, w3, b3):
    n, d1 = h.shape
    d2, d3, dout = w1.shape[1], w2.shape[1], w3.shape[1]
    blk = 1000

    def body(h_ref, w1_ref, b1_ref, w2_ref, b2_ref, w3_ref, b3_ref, o_ref):
        t = jnp.maximum(jnp.dot(h_ref[...], w1_ref[...],
                                preferred_element_type=jnp.float32, precision=lax.Precision.HIGHEST) + b1_ref[...], 0.0)
        t = jnp.maximum(jnp.dot(t, w2_ref[...],
                                preferred_element_type=jnp.float32, precision=lax.Precision.HIGHEST) + b2_ref[...], 0.0)
        o_ref[...] = jnp.dot(t, w3_ref[...],
                             preferred_element_type=jnp.float32, precision=lax.Precision.HIGHEST) + b3_ref[...]

    return pl.pallas_call(
        body,
        grid=(n // blk,),
        in_specs=[
            pl.BlockSpec((blk, d1), lambda i: (i, 0)),
            pl.BlockSpec((d1, d2), lambda i: (0, 0)),
            pl.BlockSpec((1, d2), lambda i: (0, 0)),
            pl.BlockSpec((d2, d3), lambda i: (0, 0)),
            pl.BlockSpec((1, d3), lambda i: (0, 0)),
            pl.BlockSpec((d3, dout), lambda i: (0, 0)),
            pl.BlockSpec((1, dout), lambda i: (0, 0)),
        ],
        out_specs=pl.BlockSpec((blk, dout), lambda i: (i, 0)),
        out_shape=jax.ShapeDtypeStruct((n, dout), jnp.float32),
    )(h, w1, b1.reshape(1, d2), w2, b2.reshape(1, d3), w3, b3.reshape(1, dout))


def kernel(x, edge_index, edge_attr, params):
    n, nfeat = x.shape
    e_total = edge_attr.shape[0]
    src = edge_index[0].astype(jnp.int32).reshape(e_total // CHUNK, CHUNK)
    dst = edge_index[1].astype(jnp.int32).reshape(e_total // CHUNK, CHUNK)

    d0 = 16
    h = jnp.pad(x, ((0, 0), (0, d0 - nfeat)))

    edge16 = _make_edge_aggr(n, e_total, d0)
    edge128 = _make_edge_aggr(n, e_total, 128)

    for i in range(6):
        p = params["conv%d" % i]
        if i == 0:
            we = jnp.pad(p["we"], ((0, 0), (0, d0 - nfeat)))
            be = jnp.pad(p["be"], (0, d0 - nfeat))
            w1 = jnp.pad(p["w1"], ((0, d0 - nfeat), (0, 0)))
            ek = edge16
        else:
            we, be, w1 = p["we"], p["be"], p["w1"]
            ek = edge128
        n_pad = ((n + NS * CHUNK - 1) // (NS * CHUNK)) * (NS * CHUNK)
        e = _edge_bias(edge_attr, we, be)
        ag = ek(h, src, dst, e)
        a0, a1 = ag[:n], ag[n_pad:n_pad + n]
        h = _node_mlp(h, a0, a1, w1, p["b1"], p["w2"], p["b2"])

    return _head(h, params["lin1_w"], params["lin1_b"],
                 params["lin2_w"], params["lin2_b"],
                 params["lin3_w"], params["lin3_b"])
